# Initial kernel scaffold; baseline (speedup 1.0000x reference)
import jax
import jax.numpy as jnp
from jax.experimental import pallas as pl


def kernel(x, edge_index, edge_attr, U0_w, U0_b, gru0_wih, gru0_whh, gru0_bih, gru0_bhh, U1_w, U1_b, gru1_wih, gru1_whh, gru1_bih, gru1_bhh, U2_w, U2_b, gru2_wih, gru2_whh, gru2_bih, gru2_bhh):
    raise NotImplementedError("write your pallas kernel here")


# speedup vs baseline: 7.4225x; 7.4225x over previous
import functools

import jax
import jax.numpy as jnp
from jax import lax
from jax.experimental import pallas as pl
from jax.experimental.pallas import tpu as pltpu
from jax.experimental.pallas import tpu_sc as plsc

N = 10000
E = 320000
D = 128
DE = 16

NC = 2
NS = 16
NW = NC * NS
C = 128
SUP = 8
EPAD = 320512
ROWS = EPAD // C
ST = ROWS // SUP
NPAD = 10240
ROWS_PER_TILE = NPAD // NS
TRASH = NPAD - 1

@functools.cache
def _mesh():
    return plsc.VectorSubcoreMesh(
        core_axis_name="c", subcore_axis_name="s", num_cores=NC, num_subcores=NS
    )


def _edge_loop(wid, body):
    ntrips = (ST - 1 - wid) // NW + 1

    def trip(i, carry):
        body(wid + i * NW)
        return carry

    lax.fori_loop(0, ntrips, trip, 0)


def _sc_round0_body(src_hbm, dst_hbm, h_hbm, ea_hbm, z128_hbm, z16_hbm,
                    g_out, es_out, dw_out,
                    idx_s, idx_d, rows, ea_rows, ones_v, acc, acc_es, acc_dw,
                    sem):
    c = lax.axis_index("c")
    s = lax.axis_index("s")
    wid = s * NC + c
    base = s * ROWS_PER_TILE

    pltpu.sync_copy(z128_hbm.at[pl.ds(base, ROWS_PER_TILE)],
                    acc.at[pl.ds(base, ROWS_PER_TILE)])
    pltpu.sync_copy(z16_hbm.at[pl.ds(base, ROWS_PER_TILE)],
                    acc_es.at[pl.ds(base, ROWS_PER_TILE)])
    pltpu.sync_copy(z16_hbm.at[pl.ds(base, ROWS_PER_TILE)],
                    acc_dw.at[pl.ds(base, ROWS_PER_TILE)])

    def fill_ones(i, carry):
        ones_v[i, :] = jnp.ones((16,), jnp.float32)
        return carry

    lax.fori_loop(0, C, fill_ones, 0)
    plsc.subcore_barrier()

    def body(sr):
        r0 = sr * SUP
        pltpu.sync_copy(src_hbm.at[pl.ds(r0, SUP)], idx_s)
        pltpu.sync_copy(dst_hbm.at[pl.ds(r0, SUP)], idx_d)
        for j in range(SUP):
            pltpu.sync_copy(ea_hbm.at[pl.ds((r0 + j) * C, C)], ea_rows)
            pltpu.async_copy(h_hbm.at[idx_s.at[j]], rows, sem).wait()
            pltpu.sync_copy(rows, acc.at[idx_d.at[j]], add=True)
            pltpu.sync_copy(ea_rows, acc_es.at[idx_d.at[j]], add=True)
            pltpu.sync_copy(ones_v, acc_dw.at[idx_d.at[j]], add=True)

    _edge_loop(wid, body)
    plsc.subcore_barrier()

    pltpu.sync_copy(acc.at[pl.ds(base, ROWS_PER_TILE)],
                    g_out.at[c, pl.ds(base, ROWS_PER_TILE)])
    pltpu.sync_copy(acc_es.at[pl.ds(base, ROWS_PER_TILE)],
                    es_out.at[c, pl.ds(base, ROWS_PER_TILE)])
    pltpu.sync_copy(acc_dw.at[pl.ds(base, ROWS_PER_TILE)],
                    dw_out.at[c, pl.ds(base, ROWS_PER_TILE)])


@functools.cache
def _sc_round0():
    return pl.kernel(
        _sc_round0_body,
        out_type=(
            jax.ShapeDtypeStruct((NC, NPAD, D), jnp.float32),
            jax.ShapeDtypeStruct((NC, NPAD, DE), jnp.float32),
            jax.ShapeDtypeStruct((NC, NPAD, DE), jnp.float32),
        ),
        mesh=_mesh(),
        scratch_types=[
            pltpu.VMEM((SUP, C), jnp.int32),
            pltpu.VMEM((SUP, C), jnp.int32),
            pltpu.VMEM((C, D), jnp.float32),
            pltpu.VMEM((C, DE), jnp.float32),
            pltpu.VMEM((C, DE), jnp.float32),
            pltpu.VMEM_SHARED((NPAD, D), jnp.float32),
            pltpu.VMEM_SHARED((NPAD, DE), jnp.float32),
            pltpu.VMEM_SHARED((NPAD, DE), jnp.float32),
            pltpu.SemaphoreType.DMA,
        ],
        compiler_params=pltpu.CompilerParams(use_tc_tiling_on_sc=False),
        name="sc_gather_segsum_round0",
    )


def _sc_round_body(src_hbm, dst_hbm, h_hbm, z128_hbm,
                   g_out,
                   idx_s, idx_d, rows, acc, sem):
    c = lax.axis_index("c")
    s = lax.axis_index("s")
    wid = s * NC + c
    base = s * ROWS_PER_TILE

    pltpu.sync_copy(z128_hbm.at[pl.ds(base, ROWS_PER_TILE)],
                    acc.at[pl.ds(base, ROWS_PER_TILE)])
    plsc.subcore_barrier()

    def body(sr):
        r0 = sr * SUP
        pltpu.sync_copy(src_hbm.at[pl.ds(r0, SUP)], idx_s)
        pltpu.sync_copy(dst_hbm.at[pl.ds(r0, SUP)], idx_d)
        for j in range(SUP):
            pltpu.async_copy(h_hbm.at[idx_s.at[j]], rows, sem).wait()
            pltpu.sync_copy(rows, acc.at[idx_d.at[j]], add=True)

    _edge_loop(wid, body)
    plsc.subcore_barrier()

    pltpu.sync_copy(acc.at[pl.ds(base, ROWS_PER_TILE)],
                    g_out.at[c, pl.ds(base, ROWS_PER_TILE)])


@functools.cache
def _sc_round():
    return pl.kernel(
        _sc_round_body,
        out_type=jax.ShapeDtypeStruct((NC, NPAD, D), jnp.float32),
        mesh=_mesh(),
        scratch_types=[
            pltpu.VMEM((SUP, C), jnp.int32),
            pltpu.VMEM((SUP, C), jnp.int32),
            pltpu.VMEM((C, D), jnp.float32),
            pltpu.VMEM_SHARED((NPAD, D), jnp.float32),
            pltpu.SemaphoreType.DMA,
        ],
        compiler_params=pltpu.CompilerParams(use_tc_tiling_on_sc=False),
        name="sc_gather_segsum",
    )


BN = 2000


def _tc_round_body(h_ref, hr_ref, g_ref, es_ref, dw_ref, uw_ref, ub_ref,
                   wih_ref, whh_ref, bih_ref, bhh_ref, out_ref):
    f32 = jnp.float32
    hi = lax.Precision.HIGHEST
    h = h_ref[...]
    hr = hr_ref[...]
    g = g_ref[0] + g_ref[1]
    es = es_ref[0] + es_ref[1]
    deg = dw_ref[0, :, 0:1] + dw_ref[1, :, 0:1]

    wd = uw_ref[:, 0:D]
    ws = uw_ref[:, D:2 * D]
    we = uw_ref[:, 2 * D:2 * D + DE]
    dn = (((1,), (1,)), ((), ()))
    agg = (deg * (lax.dot_general(hr, wd, dn, preferred_element_type=f32,
                                  precision=hi)
                  + ub_ref[...])
           + lax.dot_general(g, ws, dn, preferred_element_type=f32,
                             precision=hi)
           + lax.dot_general(es, we, dn, preferred_element_type=f32,
                             precision=hi))

    gi = lax.dot_general(agg, wih_ref[...], dn, preferred_element_type=f32) \
        + bih_ref[...]
    gh = lax.dot_general(h, whh_ref[...], dn, preferred_element_type=f32) \
        + bhh_ref[...]
    r = jax.nn.sigmoid(gi[:, 0:D] + gh[:, 0:D])
    z = jax.nn.sigmoid(gi[:, D:2 * D] + gh[:, D:2 * D])
    n = jnp.tanh(gi[:, 2 * D:3 * D] + r * gh[:, 2 * D:3 * D])
    out_ref[...] = (1.0 - z) * n + z * h


def _tc_round(h, hr, g, es, dw, uw, ub, wih, whh, bih, bhh):
    grid = (N // BN,)
    full = lambda i: (0, 0)
    return pl.pallas_call(
        _tc_round_body,
        grid=grid,
        in_specs=[
            pl.BlockSpec((BN, D), lambda i: (i, 0)),
            pl.BlockSpec((BN, D), lambda i: (i, 0)),
            pl.BlockSpec((NC, BN, D), lambda i: (0, i, 0)),
            pl.BlockSpec((NC, BN, DE), lambda i: (0, i, 0)),
            pl.BlockSpec((NC, BN, DE), lambda i: (0, i, 0)),
            pl.BlockSpec((D, 2 * D + DE), full),
            pl.BlockSpec((1, D), full),
            pl.BlockSpec((3 * D, D), full),
            pl.BlockSpec((3 * D, D), full),
            pl.BlockSpec((1, 3 * D), full),
            pl.BlockSpec((1, 3 * D), full),
        ],
        out_specs=pl.BlockSpec((BN, D), lambda i: (i, 0)),
        out_shape=jax.ShapeDtypeStruct((N, D), jnp.float32),
        compiler_params=pltpu.CompilerParams(
            dimension_semantics=("arbitrary",),
        ),
        name="tc_gru_round",
    )(h, hr, g, es, dw, uw, ub, wih, whh, bih, bhh)


@jax.jit
def kernel(x, edge_index, edge_attr,
           U0_w, U0_b, gru0_wih, gru0_whh, gru0_bih, gru0_bhh,
           U1_w, U1_b, gru1_wih, gru1_whh, gru1_bih, gru1_bhh,
           U2_w, U2_b, gru2_wih, gru2_whh, gru2_bih, gru2_bhh):
    npad = EPAD - E
    src2d = jnp.concatenate(
        [edge_index[0], jnp.zeros((npad,), jnp.int32)]).reshape(ROWS, C)
    dst2d = jnp.concatenate(
        [edge_index[1], jnp.full((npad,), TRASH, jnp.int32)]).reshape(ROWS, C)
    ea_pad = jnp.concatenate(
        [lax.reduce_precision(edge_attr, 8, 7),
         jnp.zeros((npad, DE), jnp.float32)], axis=0)
    z128 = jnp.zeros((NPAD, D), jnp.float32)
    z16 = jnp.zeros((NPAD, DE), jnp.float32)

    layers = [
        (U0_w, U0_b, gru0_wih, gru0_whh, gru0_bih, gru0_bhh),
        (U1_w, U1_b, gru1_wih, gru1_whh, gru1_bih, gru1_bhh),
        (U2_w, U2_b, gru2_wih, gru2_whh, gru2_bih, gru2_bhh),
    ]

    h = x
    hr = lax.reduce_precision(h, 8, 7)
    g, es, dw = _sc_round0()(src2d, dst2d, hr, ea_pad, z128, z16)
    for k, (uw, ub, wih, whh, bih, bhh) in enumerate(layers):
        if k > 0:
            hr = lax.reduce_precision(h, 8, 7)
            g = _sc_round()(src2d, dst2d, hr, z128)
        h = _tc_round(h, hr, g, es, dw,
                      lax.reduce_precision(uw, 8, 7),
                      ub.reshape(1, D),
                      wih, whh, bih.reshape(1, 3 * D), bhh.reshape(1, 3 * D))
    return h


# speedup vs baseline: 10.4823x; 1.4122x over previous
import functools

import jax
import jax.numpy as jnp
from jax import lax
from jax.experimental import pallas as pl
from jax.experimental.pallas import tpu as pltpu
from jax.experimental.pallas import tpu_sc as plsc

N = 10000
E = 320000
D = 128
DE = 16

NC = 2
NS = 16
NW = NC * NS
C = 128
SUP = 8
ROWS = E // C
ST_FULL = ROWS // SUP
TAIL0 = ST_FULL * SUP
NTAIL = ROWS - TAIL0
TAILW = 24
RPT = N // NS


@functools.cache
def _mesh():
    return plsc.VectorSubcoreMesh(
        core_axis_name="c", subcore_axis_name="s", num_cores=NC, num_subcores=NS
    )


def _worker_id():
    return lax.axis_index("s") * NC + lax.axis_index("c")


def _ntrips(wid):
    return (ST_FULL - 1 - wid) // NW + 1


def _sc_round_body(ei_hbm, h_hbm, z128_hbm, g_out,
                   idx, rows0, rows1, acc, sg0, sg1, ss0, ss1):
    c = lax.axis_index("c")
    s = lax.axis_index("s")
    wid = _worker_id()
    base = s * RPT

    pltpu.sync_copy(z128_hbm.at[pl.ds(base, RPT)], acc.at[pl.ds(base, RPT)])
    plsc.subcore_barrier()

    @pl.when((wid >= TAILW) & (wid < TAILW + NTAIL))
    def _():
        r = TAIL0 + wid - TAILW
        pltpu.sync_copy(ei_hbm.at[:, pl.ds(r, 1), :], idx.at[:, 0:1, :])
        pltpu.async_copy(h_hbm.at[idx.at[0, 0]], rows0, sg0).wait()
        pltpu.sync_copy(rows0, acc.at[idx.at[1, 0]], add=True)

    rows = [rows0, rows1]
    sg = [sg0, sg1]
    ss = [ss0, ss1]

    def trip(i, carry):
        r0 = (wid + i * NW) * SUP
        pltpu.sync_copy(ei_hbm.at[:, pl.ds(r0, SUP), :], idx)
        dg = [None] * SUP
        dscat = [None] * SUP
        dg[0] = pltpu.async_copy(h_hbm.at[idx.at[0, 0]], rows[0], sg[0])
        for j in range(SUP):
            b = j & 1
            dg[j].wait()
            if j + 1 < SUP:
                if j >= 1:
                    dscat[j - 1].wait()
                dg[j + 1] = pltpu.async_copy(
                    h_hbm.at[idx.at[0, j + 1]], rows[1 - b], sg[1 - b])
            dscat[j] = pltpu.async_copy(
                rows[b], acc.at[idx.at[1, j]], ss[b], add=True)
        dscat[SUP - 2].wait()
        dscat[SUP - 1].wait()
        return carry

    lax.fori_loop(0, _ntrips(wid), trip, 0)
    plsc.subcore_barrier()

    pltpu.sync_copy(acc.at[pl.ds(base, RPT)], g_out.at[c, pl.ds(base, RPT)])


@functools.cache
def _sc_round():
    return pl.kernel(
        _sc_round_body,
        out_type=jax.ShapeDtypeStruct((NC, N, D), jnp.float32),
        mesh=_mesh(),
        scratch_types=[
            pltpu.VMEM((2, SUP, C), jnp.int32),
            pltpu.VMEM((C, D), jnp.float32),
            pltpu.VMEM((C, D), jnp.float32),
            pltpu.VMEM_SHARED((N, D), jnp.float32),
            pltpu.SemaphoreType.DMA,
            pltpu.SemaphoreType.DMA,
            pltpu.SemaphoreType.DMA,
            pltpu.SemaphoreType.DMA,
        ],
        compiler_params=pltpu.CompilerParams(use_tc_tiling_on_sc=False),
        name="sc_gather_segsum",
    )


def _sc_round0_body(ei_hbm, h_hbm, ea_hbm, z128_hbm, z16_hbm,
                    g_out, es_out, dw_out,
                    idx, rows0, ea_v, ones_v, acc, acc_es, acc_dw,
                    sg0, se0, so0):
    c = lax.axis_index("c")
    s = lax.axis_index("s")
    wid = _worker_id()
    base = s * RPT

    pltpu.sync_copy(z128_hbm.at[pl.ds(base, RPT)], acc.at[pl.ds(base, RPT)])
    pltpu.sync_copy(z16_hbm.at[pl.ds(base, RPT)], acc_es.at[pl.ds(base, RPT)])
    pltpu.sync_copy(z16_hbm.at[pl.ds(base, RPT)], acc_dw.at[pl.ds(base, RPT)])

    def fill_ones(i, carry):
        ones_v[i, :] = jnp.ones((16,), jnp.float32)
        return carry

    lax.fori_loop(0, C, fill_ones, 0)
    plsc.subcore_barrier()

    @pl.when((wid >= TAILW) & (wid < TAILW + NTAIL))
    def _():
        r = TAIL0 + wid - TAILW
        pltpu.sync_copy(ei_hbm.at[:, pl.ds(r, 1), :], idx.at[:, 0:1, :])
        pltpu.sync_copy(ea_hbm.at[pl.ds(r * C, C)], ea_v)
        pltpu.async_copy(h_hbm.at[idx.at[0, 0]], rows0, sg0).wait()
        pltpu.sync_copy(rows0, acc.at[idx.at[1, 0]], add=True)
        pltpu.sync_copy(ea_v, acc_es.at[idx.at[1, 0]], add=True)
        pltpu.sync_copy(ones_v, acc_dw.at[idx.at[1, 0]], add=True)

    def trip(i, carry):
        r0 = (wid + i * NW) * SUP
        pltpu.sync_copy(ei_hbm.at[:, pl.ds(r0, SUP), :], idx)
        de = [None] * SUP
        do = [None] * SUP
        for j in range(SUP):
            dg = pltpu.async_copy(h_hbm.at[idx.at[0, j]], rows0, sg0)
            if j >= 1:
                de[j - 1].wait()
                do[j - 1].wait()
            pltpu.sync_copy(ea_hbm.at[pl.ds((r0 + j) * C, C)], ea_v)
            dg.wait()
            de[j] = pltpu.async_copy(ea_v, acc_es.at[idx.at[1, j]], se0,
                                     add=True)
            do[j] = pltpu.async_copy(ones_v, acc_dw.at[idx.at[1, j]], so0,
                                     add=True)
            pltpu.sync_copy(rows0, acc.at[idx.at[1, j]], add=True)
        de[SUP - 1].wait()
        do[SUP - 1].wait()
        return carry

    lax.fori_loop(0, _ntrips(wid), trip, 0)
    plsc.subcore_barrier()

    pltpu.sync_copy(acc.at[pl.ds(base, RPT)], g_out.at[c, pl.ds(base, RPT)])
    pltpu.sync_copy(acc_es.at[pl.ds(base, RPT)],
                    es_out.at[c, pl.ds(base, RPT)])
    pltpu.sync_copy(acc_dw.at[pl.ds(base, RPT)],
                    dw_out.at[c, pl.ds(base, RPT)])


@functools.cache
def _sc_round0():
    return pl.kernel(
        _sc_round0_body,
        out_type=(
            jax.ShapeDtypeStruct((NC, N, D), jnp.float32),
            jax.ShapeDtypeStruct((NC, N, DE), jnp.float32),
            jax.ShapeDtypeStruct((NC, N, DE), jnp.float32),
        ),
        mesh=_mesh(),
        scratch_types=[
            pltpu.VMEM((2, SUP, C), jnp.int32),
            pltpu.VMEM((C, D), jnp.float32),
            pltpu.VMEM((C, DE), jnp.float32),
            pltpu.VMEM((C, DE), jnp.float32),
            pltpu.VMEM_SHARED((N, D), jnp.float32),
            pltpu.VMEM_SHARED((N, DE), jnp.float32),
            pltpu.VMEM_SHARED((N, DE), jnp.float32),
            pltpu.SemaphoreType.DMA,
            pltpu.SemaphoreType.DMA,
            pltpu.SemaphoreType.DMA,
        ],
        compiler_params=pltpu.CompilerParams(use_tc_tiling_on_sc=False),
        name="sc_gather_segsum_round0",
    )


BN = 2000


def _tc_round_body(h_ref, hr_ref, g_ref, es_ref, dw_ref, uw_ref, ub_ref,
                   wih_ref, whh_ref, bih_ref, bhh_ref, out_ref):
    f32 = jnp.float32
    hi = lax.Precision.HIGHEST
    h = h_ref[...]
    hr = hr_ref[...]
    g = g_ref[0] + g_ref[1]
    es = es_ref[0] + es_ref[1]
    deg = dw_ref[0, :, 0:1] + dw_ref[1, :, 0:1]

    wd = uw_ref[:, 0:D]
    ws = uw_ref[:, D:2 * D]
    we = uw_ref[:, 2 * D:2 * D + DE]
    dn = (((1,), (1,)), ((), ()))
    agg = (deg * (lax.dot_general(hr, wd, dn, preferred_element_type=f32,
                                  precision=hi)
                  + ub_ref[...])
           + lax.dot_general(g, ws, dn, preferred_element_type=f32,
                             precision=hi)
           + lax.dot_general(es, we, dn, preferred_element_type=f32,
                             precision=hi))

    gi = lax.dot_general(agg, wih_ref[...], dn, preferred_element_type=f32) \
        + bih_ref[...]
    gh = lax.dot_general(h, whh_ref[...], dn, preferred_element_type=f32) \
        + bhh_ref[...]
    r = jax.nn.sigmoid(gi[:, 0:D] + gh[:, 0:D])
    z = jax.nn.sigmoid(gi[:, D:2 * D] + gh[:, D:2 * D])
    n = jnp.tanh(gi[:, 2 * D:3 * D] + r * gh[:, 2 * D:3 * D])
    out_ref[...] = (1.0 - z) * n + z * h


def _tc_round(h, hr, g, es, dw, uw, ub, wih, whh, bih, bhh):
    grid = (N // BN,)
    full = lambda i: (0, 0)
    return pl.pallas_call(
        _tc_round_body,
        grid=grid,
        in_specs=[
            pl.BlockSpec((BN, D), lambda i: (i, 0)),
            pl.BlockSpec((BN, D), lambda i: (i, 0)),
            pl.BlockSpec((NC, BN, D), lambda i: (0, i, 0)),
            pl.BlockSpec((NC, BN, DE), lambda i: (0, i, 0)),
            pl.BlockSpec((NC, BN, DE), lambda i: (0, i, 0)),
            pl.BlockSpec((D, 2 * D + DE), full),
            pl.BlockSpec((1, D), full),
            pl.BlockSpec((3 * D, D), full),
            pl.BlockSpec((3 * D, D), full),
            pl.BlockSpec((1, 3 * D), full),
            pl.BlockSpec((1, 3 * D), full),
        ],
        out_specs=pl.BlockSpec((BN, D), lambda i: (i, 0)),
        out_shape=jax.ShapeDtypeStruct((N, D), jnp.float32),
        compiler_params=pltpu.CompilerParams(
            dimension_semantics=("arbitrary",),
        ),
        name="tc_gru_round",
    )(h, hr, g, es, dw, uw, ub, wih, whh, bih, bhh)


@jax.jit
def kernel(x, edge_index, edge_attr,
           U0_w, U0_b, gru0_wih, gru0_whh, gru0_bih, gru0_bhh,
           U1_w, U1_b, gru1_wih, gru1_whh, gru1_bih, gru1_bhh,
           U2_w, U2_b, gru2_wih, gru2_whh, gru2_bih, gru2_bhh):
    ei = edge_index.reshape(2, ROWS, C)
    ea_r = lax.reduce_precision(edge_attr, 8, 7)
    z128 = jnp.zeros((N, D), jnp.float32)
    z16 = jnp.zeros((N, DE), jnp.float32)

    layers = [
        (U0_w, U0_b, gru0_wih, gru0_whh, gru0_bih, gru0_bhh),
        (U1_w, U1_b, gru1_wih, gru1_whh, gru1_bih, gru1_bhh),
        (U2_w, U2_b, gru2_wih, gru2_whh, gru2_bih, gru2_bhh),
    ]

    h = x
    hr = lax.reduce_precision(h, 8, 7)
    g, es, dw = _sc_round0()(ei, hr, ea_r, z128, z16)
    for k, (uw, ub, wih, whh, bih, bhh) in enumerate(layers):
        if k > 0:
            hr = lax.reduce_precision(h, 8, 7)
            g = _sc_round()(ei, hr, z128)
        h = _tc_round(h, hr, g, es, dw,
                      lax.reduce_precision(uw, 8, 7),
                      ub.reshape(1, D),
                      wih, whh, bih.reshape(1, 3 * D), bhh.reshape(1, 3 * D))
    return h


# speedup vs baseline: 10.6962x; 1.0204x over previous
import functools

import jax
import jax.numpy as jnp
from jax import lax
from jax.experimental import pallas as pl
from jax.experimental.pallas import tpu as pltpu
from jax.experimental.pallas import tpu_sc as plsc

N = 10000
E = 320000
D = 128
DE = 16

NC = 2
NS = 16
NW = NC * NS
C = 128
SUP = 16
SUP0 = 8
ROWS = E // C
ST_FULL = ROWS // SUP
ST0_FULL = ROWS // SUP0
TAIL0 = ST_FULL * SUP
NTAIL = ROWS - TAIL0
TAILW = 24
RPT = N // NS
DW = 2 * DE


@functools.cache
def _mesh():
    return plsc.VectorSubcoreMesh(
        core_axis_name="c", subcore_axis_name="s", num_cores=NC, num_subcores=NS
    )


def _worker_id():
    return lax.axis_index("s") * NC + lax.axis_index("c")


def _ntrips(wid, st):
    return (st - 1 - wid) // NW + 1


def _sc_round_body(ei_hbm, h_hbm, z128_hbm, g_out,
                   idx, rows0, rows1, acc, sg0, sg1, ss0, ss1):
    c = lax.axis_index("c")
    s = lax.axis_index("s")
    wid = _worker_id()
    base = s * RPT

    pltpu.sync_copy(z128_hbm.at[pl.ds(base, RPT)], acc.at[pl.ds(base, RPT)])
    plsc.subcore_barrier()

    @pl.when((wid >= TAILW) & (wid < TAILW + NTAIL))
    def _():
        r = TAIL0 + wid - TAILW
        pltpu.sync_copy(ei_hbm.at[:, pl.ds(r, 1), :], idx.at[:, 0:1, :])
        pltpu.async_copy(h_hbm.at[idx.at[0, 0]], rows0, sg0).wait()
        pltpu.sync_copy(rows0, acc.at[idx.at[1, 0]], add=True)

    rows = [rows0, rows1]
    sg = [sg0, sg1]
    ss = [ss0, ss1]

    def trip(i, carry):
        r0 = (wid + i * NW) * SUP
        pltpu.sync_copy(ei_hbm.at[:, pl.ds(r0, SUP), :], idx)
        dg = [None] * SUP
        dscat = [None] * SUP
        dg[0] = pltpu.async_copy(h_hbm.at[idx.at[0, 0]], rows[0], sg[0])
        for j in range(SUP):
            b = j & 1
            dg[j].wait()
            if j + 1 < SUP:
                if j >= 1:
                    dscat[j - 1].wait()
                dg[j + 1] = pltpu.async_copy(
                    h_hbm.at[idx.at[0, j + 1]], rows[1 - b], sg[1 - b])
            dscat[j] = pltpu.async_copy(
                rows[b], acc.at[idx.at[1, j]], ss[b], add=True)
        dscat[SUP - 2].wait()
        dscat[SUP - 1].wait()
        return carry

    lax.fori_loop(0, _ntrips(wid, ST_FULL), trip, 0)
    plsc.subcore_barrier()

    pltpu.sync_copy(acc.at[pl.ds(base, RPT)], g_out.at[c, pl.ds(base, RPT)])


@functools.cache
def _sc_round():
    return pl.kernel(
        _sc_round_body,
        out_type=jax.ShapeDtypeStruct((NC, N, D), jnp.float32),
        mesh=_mesh(),
        scratch_types=[
            pltpu.VMEM((2, SUP, C), jnp.int32),
            pltpu.VMEM((C, D), jnp.float32),
            pltpu.VMEM((C, D), jnp.float32),
            pltpu.VMEM_SHARED((N, D), jnp.float32),
            pltpu.SemaphoreType.DMA,
            pltpu.SemaphoreType.DMA,
            pltpu.SemaphoreType.DMA,
            pltpu.SemaphoreType.DMA,
        ],
        compiler_params=pltpu.CompilerParams(use_tc_tiling_on_sc=False),
        name="sc_gather_segsum",
    )


def _sc_round0_body(ei_hbm, h_hbm, ea_hbm, z128_hbm,
                    g_out, esdw_out,
                    idx, rows0, esdw_v, acc, acc_esdw,
                    sg0, se0):
    c = lax.axis_index("c")
    s = lax.axis_index("s")
    wid = _worker_id()
    base = s * RPT

    pltpu.sync_copy(z128_hbm.at[pl.ds(base, RPT)], acc.at[pl.ds(base, RPT)])
    pltpu.sync_copy(z128_hbm.at[pl.ds(base, RPT), 0:DW],
                    acc_esdw.at[pl.ds(base, RPT)])

    def fill_ones(i, carry):
        esdw_v[i, DE:DW] = jnp.ones((DE,), jnp.float32)
        return carry

    lax.fori_loop(0, C, fill_ones, 0)
    plsc.subcore_barrier()

    @pl.when((wid >= TAILW) & (wid < TAILW + NTAIL))
    def _():
        r = TAIL0 + wid - TAILW
        pltpu.sync_copy(ei_hbm.at[:, pl.ds(r, 1), :], idx.at[:, 0:1, :])
        pltpu.sync_copy(ea_hbm.at[pl.ds(r * C, C)], esdw_v.at[:, 0:DE])
        pltpu.async_copy(h_hbm.at[idx.at[0, 0]], rows0, sg0).wait()
        pltpu.sync_copy(rows0, acc.at[idx.at[1, 0]], add=True)
        pltpu.sync_copy(esdw_v, acc_esdw.at[idx.at[1, 0]], add=True)

    def trip(i, carry):
        r0 = (wid + i * NW) * SUP0
        pltpu.sync_copy(ei_hbm.at[:, pl.ds(r0, SUP0), :],
                        idx.at[:, 0:SUP0, :])
        de = [None] * SUP0
        for j in range(SUP0):
            dg = pltpu.async_copy(h_hbm.at[idx.at[0, j]], rows0, sg0)
            if j >= 1:
                de[j - 1].wait()
            pltpu.sync_copy(ea_hbm.at[pl.ds((r0 + j) * C, C)],
                            esdw_v.at[:, 0:DE])
            dg.wait()
            de[j] = pltpu.async_copy(esdw_v, acc_esdw.at[idx.at[1, j]], se0,
                                     add=True)
            pltpu.sync_copy(rows0, acc.at[idx.at[1, j]], add=True)
        de[SUP0 - 1].wait()
        return carry

    lax.fori_loop(0, _ntrips(wid, ST0_FULL), trip, 0)
    plsc.subcore_barrier()

    pltpu.sync_copy(acc.at[pl.ds(base, RPT)], g_out.at[c, pl.ds(base, RPT)])
    pltpu.sync_copy(acc_esdw.at[pl.ds(base, RPT)],
                    esdw_out.at[c, pl.ds(base, RPT)])


@functools.cache
def _sc_round0():
    return pl.kernel(
        _sc_round0_body,
        out_type=(
            jax.ShapeDtypeStruct((NC, N, D), jnp.float32),
            jax.ShapeDtypeStruct((NC, N, DW), jnp.float32),
        ),
        mesh=_mesh(),
        scratch_types=[
            pltpu.VMEM((2, SUP, C), jnp.int32),
            pltpu.VMEM((C, D), jnp.float32),
            pltpu.VMEM((C, DW), jnp.float32),
            pltpu.VMEM_SHARED((N, D), jnp.float32),
            pltpu.VMEM_SHARED((N, DW), jnp.float32),
            pltpu.SemaphoreType.DMA,
            pltpu.SemaphoreType.DMA,
        ],
        compiler_params=pltpu.CompilerParams(use_tc_tiling_on_sc=False),
        name="sc_gather_segsum_round0",
    )


BN = 2000


def _tc_round_body(h_ref, hr_ref, g_ref, esdw_ref, uw_ref, ub_ref,
                   wih_ref, whh_ref, bih_ref, bhh_ref, out_ref):
    f32 = jnp.float32
    hi = lax.Precision.HIGHEST
    h = h_ref[...]
    hr = hr_ref[...]
    g = g_ref[0] + g_ref[1]
    esdw = esdw_ref[0] + esdw_ref[1]
    es = esdw[:, 0:DE]
    deg = esdw[:, DE:DE + 1]

    wd = uw_ref[:, 0:D]
    ws = uw_ref[:, D:2 * D]
    we = uw_ref[:, 2 * D:2 * D + DE]
    dn = (((1,), (1,)), ((), ()))
    agg = (deg * (lax.dot_general(hr, wd, dn, preferred_element_type=f32,
                                  precision=hi)
                  + ub_ref[...])
           + lax.dot_general(g, ws, dn, preferred_element_type=f32,
                             precision=hi)
           + lax.dot_general(es, we, dn, preferred_element_type=f32,
                             precision=hi))

    gi = lax.dot_general(agg, wih_ref[...], dn, preferred_element_type=f32) \
        + bih_ref[...]
    gh = lax.dot_general(h, whh_ref[...], dn, preferred_element_type=f32) \
        + bhh_ref[...]
    r = jax.nn.sigmoid(gi[:, 0:D] + gh[:, 0:D])
    z = jax.nn.sigmoid(gi[:, D:2 * D] + gh[:, D:2 * D])
    n = jnp.tanh(gi[:, 2 * D:3 * D] + r * gh[:, 2 * D:3 * D])
    out_ref[...] = (1.0 - z) * n + z * h


def _tc_round(h, hr, g, esdw, uw, ub, wih, whh, bih, bhh):
    grid = (N // BN,)
    full = lambda i: (0, 0)
    return pl.pallas_call(
        _tc_round_body,
        grid=grid,
        in_specs=[
            pl.BlockSpec((BN, D), lambda i: (i, 0)),
            pl.BlockSpec((BN, D), lambda i: (i, 0)),
            pl.BlockSpec((NC, BN, D), lambda i: (0, i, 0)),
            pl.BlockSpec((NC, BN, DW), lambda i: (0, i, 0)),
            pl.BlockSpec((D, 2 * D + DE), full),
            pl.BlockSpec((1, D), full),
            pl.BlockSpec((3 * D, D), full),
            pl.BlockSpec((3 * D, D), full),
            pl.BlockSpec((1, 3 * D), full),
            pl.BlockSpec((1, 3 * D), full),
        ],
        out_specs=pl.BlockSpec((BN, D), lambda i: (i, 0)),
        out_shape=jax.ShapeDtypeStruct((N, D), jnp.float32),
        compiler_params=pltpu.CompilerParams(
            dimension_semantics=("arbitrary",),
        ),
        name="tc_gru_round",
    )(h, hr, g, esdw, uw, ub, wih, whh, bih, bhh)


@jax.jit
def kernel(x, edge_index, edge_attr,
           U0_w, U0_b, gru0_wih, gru0_whh, gru0_bih, gru0_bhh,
           U1_w, U1_b, gru1_wih, gru1_whh, gru1_bih, gru1_bhh,
           U2_w, U2_b, gru2_wih, gru2_whh, gru2_bih, gru2_bhh):
    ei = edge_index.reshape(2, ROWS, C)
    ea_r = lax.reduce_precision(edge_attr, 8, 7)
    z128 = jnp.zeros((N, D), jnp.float32)

    layers = [
        (U0_w, U0_b, gru0_wih, gru0_whh, gru0_bih, gru0_bhh),
        (U1_w, U1_b, gru1_wih, gru1_whh, gru1_bih, gru1_bhh),
        (U2_w, U2_b, gru2_wih, gru2_whh, gru2_bih, gru2_bhh),
    ]

    h = x
    hr = lax.reduce_precision(h, 8, 7)
    g, esdw = _sc_round0()(ei, hr, ea_r, z128)
    for k, (uw, ub, wih, whh, bih, bhh) in enumerate(layers):
        if k > 0:
            hr = lax.reduce_precision(h, 8, 7)
            g = _sc_round()(ei, hr, z128)
        h = _tc_round(h, hr, g, esdw,
                      lax.reduce_precision(uw, 8, 7),
                      ub.reshape(1, D),
                      wih, whh, bih.reshape(1, 3 * D), bhh.reshape(1, 3 * D))
    return h


# speedup vs baseline: 10.7616x; 1.0061x over previous
import functools

import jax
import jax.numpy as jnp
from jax import lax
from jax.experimental import pallas as pl
from jax.experimental.pallas import tpu as pltpu
from jax.experimental.pallas import tpu_sc as plsc

N = 10000
E = 320000
D = 128
DE = 16

NC = 2
NS = 16
NW = NC * NS
C = 128
SUP = 16
ROWS = E // C
ST_FULL = ROWS // SUP
TAIL0 = ST_FULL * SUP
NTAIL = ROWS - TAIL0
TAILW = 24
RPT = N // NS
DW = 2 * DE


@functools.cache
def _mesh():
    return plsc.VectorSubcoreMesh(
        core_axis_name="c", subcore_axis_name="s", num_cores=NC, num_subcores=NS
    )


def _worker_id():
    return lax.axis_index("s") * NC + lax.axis_index("c")


def _ntrips(wid, st):
    return (st - 1 - wid) // NW + 1


def _sc_round_body(ei_hbm, h_hbm, z128_hbm, g_out,
                   idx, rows0, rows1, acc, sg0, sg1, ss0, ss1):
    c = lax.axis_index("c")
    s = lax.axis_index("s")
    wid = _worker_id()
    base = s * RPT

    pltpu.sync_copy(z128_hbm.at[pl.ds(base, RPT)], acc.at[pl.ds(base, RPT)])
    plsc.subcore_barrier()

    @pl.when((wid >= TAILW) & (wid < TAILW + NTAIL))
    def _():
        r = TAIL0 + wid - TAILW
        pltpu.sync_copy(ei_hbm.at[:, pl.ds(r * C, C)], idx.at[:, 0:C])
        pltpu.async_copy(h_hbm.at[idx.at[0, 0:C]], rows0, sg0).wait()
        pltpu.sync_copy(rows0, acc.at[idx.at[1, 0:C]], add=True)

    rows = [rows0, rows1]
    sg = [sg0, sg1]
    ss = [ss0, ss1]

    def trip(i, carry):
        e0 = (wid + i * NW) * SUP * C
        pltpu.sync_copy(ei_hbm.at[:, pl.ds(e0, SUP * C)], idx)
        dg = [None] * SUP
        dscat = [None] * SUP
        dg[0] = pltpu.async_copy(h_hbm.at[idx.at[0, 0:C]], rows[0], sg[0])
        for j in range(SUP):
            b = j & 1
            dg[j].wait()
            if j + 1 < SUP:
                if j >= 1:
                    dscat[j - 1].wait()
                dg[j + 1] = pltpu.async_copy(
                    h_hbm.at[idx.at[0, pl.ds((j + 1) * C, C)]],
                    rows[1 - b], sg[1 - b])
            dscat[j] = pltpu.async_copy(
                rows[b], acc.at[idx.at[1, pl.ds(j * C, C)]], ss[b], add=True)
        dscat[SUP - 2].wait()
        dscat[SUP - 1].wait()
        return carry

    lax.fori_loop(0, _ntrips(wid, ST_FULL), trip, 0)
    plsc.subcore_barrier()

    pltpu.sync_copy(acc.at[pl.ds(base, RPT)], g_out.at[c, pl.ds(base, RPT)])


@functools.cache
def _sc_round():
    return pl.kernel(
        _sc_round_body,
        out_type=jax.ShapeDtypeStruct((NC, N, D), jnp.float32),
        mesh=_mesh(),
        scratch_types=[
            pltpu.VMEM((2, SUP * C), jnp.int32),
            pltpu.VMEM((C, D), jnp.float32),
            pltpu.VMEM((C, D), jnp.float32),
            pltpu.VMEM_SHARED((N, D), jnp.float32),
            pltpu.SemaphoreType.DMA,
            pltpu.SemaphoreType.DMA,
            pltpu.SemaphoreType.DMA,
            pltpu.SemaphoreType.DMA,
        ],
        compiler_params=pltpu.CompilerParams(use_tc_tiling_on_sc=False),
        name="sc_gather_segsum",
    )


def _sc_round0_body(ei_hbm, h_hbm, ea_hbm, z128_hbm,
                    g_out, esdw_out,
                    idx, rows0, esdw_v, acc, acc_esdw,
                    sg0, se0):
    c = lax.axis_index("c")
    s = lax.axis_index("s")
    wid = _worker_id()
    base = s * RPT

    pltpu.sync_copy(z128_hbm.at[pl.ds(base, RPT)], acc.at[pl.ds(base, RPT)])
    pltpu.sync_copy(z128_hbm.at[pl.ds(base, RPT), 0:DW],
                    acc_esdw.at[pl.ds(base, RPT)])

    def fill_ones(i, carry):
        esdw_v[i, DE:DW] = jnp.ones((DE,), jnp.float32)
        return carry

    lax.fori_loop(0, C, fill_ones, 0)
    plsc.subcore_barrier()

    @pl.when((wid >= TAILW) & (wid < TAILW + NTAIL))
    def _():
        r = TAIL0 + wid - TAILW
        pltpu.sync_copy(ei_hbm.at[:, pl.ds(r * C, C)], idx.at[:, 0:C])
        pltpu.sync_copy(ea_hbm.at[pl.ds(r * C, C)], esdw_v.at[:, 0:DE])
        pltpu.async_copy(h_hbm.at[idx.at[0, 0:C]], rows0, sg0).wait()
        pltpu.sync_copy(rows0, acc.at[idx.at[1, 0:C]], add=True)
        pltpu.sync_copy(esdw_v, acc_esdw.at[idx.at[1, 0:C]], add=True)

    def trip(i, carry):
        e0 = (wid + i * NW) * SUP * C
        pltpu.sync_copy(ei_hbm.at[:, pl.ds(e0, SUP * C)], idx)
        de = [None] * SUP
        for j in range(SUP):
            dg = pltpu.async_copy(h_hbm.at[idx.at[0, pl.ds(j * C, C)]],
                                  rows0, sg0)
            if j >= 1:
                de[j - 1].wait()
            pltpu.sync_copy(ea_hbm.at[pl.ds(e0 + j * C, C)],
                            esdw_v.at[:, 0:DE])
            dg.wait()
            de[j] = pltpu.async_copy(esdw_v,
                                     acc_esdw.at[idx.at[1, pl.ds(j * C, C)]],
                                     se0, add=True)
            pltpu.sync_copy(rows0, acc.at[idx.at[1, pl.ds(j * C, C)]],
                            add=True)
        de[SUP - 1].wait()
        return carry

    lax.fori_loop(0, _ntrips(wid, ST_FULL), trip, 0)
    plsc.subcore_barrier()

    pltpu.sync_copy(acc.at[pl.ds(base, RPT)], g_out.at[c, pl.ds(base, RPT)])
    pltpu.sync_copy(acc_esdw.at[pl.ds(base, RPT)],
                    esdw_out.at[c, pl.ds(base, RPT)])


@functools.cache
def _sc_round0():
    return pl.kernel(
        _sc_round0_body,
        out_type=(
            jax.ShapeDtypeStruct((NC, N, D), jnp.float32),
            jax.ShapeDtypeStruct((NC, N, DW), jnp.float32),
        ),
        mesh=_mesh(),
        scratch_types=[
            pltpu.VMEM((2, SUP * C), jnp.int32),
            pltpu.VMEM((C, D), jnp.float32),
            pltpu.VMEM((C, DW), jnp.float32),
            pltpu.VMEM_SHARED((N, D), jnp.float32),
            pltpu.VMEM_SHARED((N, DW), jnp.float32),
            pltpu.SemaphoreType.DMA,
            pltpu.SemaphoreType.DMA,
        ],
        compiler_params=pltpu.CompilerParams(use_tc_tiling_on_sc=False),
        name="sc_gather_segsum_round0",
    )


BN = 2000


def _tc_round_body(h_ref, hr_ref, g_ref, esdw_ref, uw_ref, ub_ref,
                   wih_ref, whh_ref, bih_ref, bhh_ref, out_ref):
    f32 = jnp.float32
    hi = lax.Precision.HIGHEST
    h = h_ref[...]
    hr = hr_ref[...]
    g = g_ref[0] + g_ref[1]
    esdw = esdw_ref[0] + esdw_ref[1]
    es = esdw[:, 0:DE]
    deg = esdw[:, DE:DE + 1]

    wd = uw_ref[:, 0:D]
    ws = uw_ref[:, D:2 * D]
    we = uw_ref[:, 2 * D:2 * D + DE]
    dn = (((1,), (1,)), ((), ()))
    agg = (deg * (lax.dot_general(hr, wd, dn, preferred_element_type=f32,
                                  precision=hi)
                  + ub_ref[...])
           + lax.dot_general(g, ws, dn, preferred_element_type=f32,
                             precision=hi)
           + lax.dot_general(es, we, dn, preferred_element_type=f32,
                             precision=hi))

    gi = lax.dot_general(agg, wih_ref[...], dn, preferred_element_type=f32) \
        + bih_ref[...]
    gh = lax.dot_general(h, whh_ref[...], dn, preferred_element_type=f32) \
        + bhh_ref[...]
    r = jax.nn.sigmoid(gi[:, 0:D] + gh[:, 0:D])
    z = jax.nn.sigmoid(gi[:, D:2 * D] + gh[:, D:2 * D])
    n = jnp.tanh(gi[:, 2 * D:3 * D] + r * gh[:, 2 * D:3 * D])
    out_ref[...] = (1.0 - z) * n + z * h


def _tc_round(h, hr, g, esdw, uw, ub, wih, whh, bih, bhh):
    grid = (N // BN,)
    full = lambda i: (0, 0)
    return pl.pallas_call(
        _tc_round_body,
        grid=grid,
        in_specs=[
            pl.BlockSpec((BN, D), lambda i: (i, 0)),
            pl.BlockSpec((BN, D), lambda i: (i, 0)),
            pl.BlockSpec((NC, BN, D), lambda i: (0, i, 0)),
            pl.BlockSpec((NC, BN, DW), lambda i: (0, i, 0)),
            pl.BlockSpec((D, 2 * D + DE), full),
            pl.BlockSpec((1, D), full),
            pl.BlockSpec((3 * D, D), full),
            pl.BlockSpec((3 * D, D), full),
            pl.BlockSpec((1, 3 * D), full),
            pl.BlockSpec((1, 3 * D), full),
        ],
        out_specs=pl.BlockSpec((BN, D), lambda i: (i, 0)),
        out_shape=jax.ShapeDtypeStruct((N, D), jnp.float32),
        compiler_params=pltpu.CompilerParams(
            dimension_semantics=("arbitrary",),
        ),
        name="tc_gru_round",
    )(h, hr, g, esdw, uw, ub, wih, whh, bih, bhh)


@jax.jit
def kernel(x, edge_index, edge_attr,
           U0_w, U0_b, gru0_wih, gru0_whh, gru0_bih, gru0_bhh,
           U1_w, U1_b, gru1_wih, gru1_whh, gru1_bih, gru1_bhh,
           U2_w, U2_b, gru2_wih, gru2_whh, gru2_bih, gru2_bhh):
    ei = edge_index
    ea_r = lax.reduce_precision(edge_attr, 8, 7)
    z128 = jnp.zeros((N, D), jnp.float32)

    layers = [
        (U0_w, U0_b, gru0_wih, gru0_whh, gru0_bih, gru0_bhh),
        (U1_w, U1_b, gru1_wih, gru1_whh, gru1_bih, gru1_bhh),
        (U2_w, U2_b, gru2_wih, gru2_whh, gru2_bih, gru2_bhh),
    ]

    h = x
    hr = lax.reduce_precision(h, 8, 7)
    g, esdw = _sc_round0()(ei, hr, ea_r, z128)
    for k, (uw, ub, wih, whh, bih, bhh) in enumerate(layers):
        if k > 0:
            hr = lax.reduce_precision(h, 8, 7)
            g = _sc_round()(ei, hr, z128)
        h = _tc_round(h, hr, g, esdw,
                      lax.reduce_precision(uw, 8, 7),
                      ub.reshape(1, D),
                      wih, whh, bih.reshape(1, 3 * D), bhh.reshape(1, 3 * D))
    return h
